# Initial kernel scaffold; baseline (speedup 1.0000x reference)
import functools

import jax
import jax.numpy as jnp
from jax import lax
from jax.experimental import pallas as pl
from jax.experimental.pallas import tpu as pltpu
from jax.experimental.pallas import tpu_sc as plsc

N = 10000
E = 320000
NC = 2
NS = 16
NW = NC * NS
N_PAD = 10240
RPS = N_PAD // NS
EPW = E // NW
CHUNK = 125
NCHUNK = EPW // CHUNK

_mesh = lambda: plsc.VectorSubcoreMesh(core_axis_name="c", subcore_axis_name="s")


@functools.partial(
    pl.kernel,
    out_type=jax.ShapeDtypeStruct((NC, N_PAD), jnp.float32),
    mesh=_mesh(),
    scratch_types=[
        pltpu.VMEM_SHARED((N_PAD,), jnp.float32),
        pltpu.VMEM((NCHUNK, CHUNK), jnp.int32),
        pltpu.VMEM((RPS,), jnp.float32),
        pltpu.VMEM((128,), jnp.float32),
    ],
)
def _deg(dst_hbm, out_hbm, deg_sh, didx_v, zeros_v, ones_v):
    c = lax.axis_index("c")
    s = lax.axis_index("s")
    wid = c * NS + s
    for i in range(RPS // 16):
        zeros_v[pl.ds(i * 16, 16)] = jnp.zeros((16,), jnp.float32)
    for i in range(8):
        ones_v[pl.ds(i * 16, 16)] = jnp.ones((16,), jnp.float32)
    pltpu.sync_copy(zeros_v, deg_sh.at[pl.ds(s * RPS, RPS)])
    pltpu.sync_copy(dst_hbm.at[wid], didx_v)
    plsc.subcore_barrier()

    def step(j, _):
        pltpu.sync_copy(ones_v.at[pl.ds(0, CHUNK)], deg_sh.at[didx_v.at[j]], add=True)
        return ()

    lax.fori_loop(0, NCHUNK, step, ())
    plsc.subcore_barrier()

    @pl.when(s == 0)
    def _():
        pltpu.sync_copy(deg_sh, out_hbm.at[c])


def _make_prop(D):
    @functools.partial(
        pl.kernel,
        out_type=jax.ShapeDtypeStruct((NC, N_PAD, D), jnp.float32),
        mesh=_mesh(),
        scratch_types=[
            pltpu.VMEM_SHARED((N_PAD, D), jnp.float32),
            pltpu.VMEM_SHARED((N_PAD, D), jnp.float32),
            pltpu.VMEM((NCHUNK, CHUNK), jnp.int32),
            pltpu.VMEM((NCHUNK, CHUNK), jnp.int32),
            pltpu.VMEM((CHUNK, D), jnp.float32),
            pltpu.SemaphoreType.DMA,
        ],
    )
    def _prop(src_hbm, dst_hbm, table_hbm, out_hbm,
              tab_sh, acc_sh, sidx_v, didx_v, rows_v, sem):
        c = lax.axis_index("c")
        s = lax.axis_index("s")
        wid = c * NS + s
        r0 = s * RPS
        pltpu.sync_copy(table_hbm.at[pl.ds(r0, RPS)], tab_sh.at[pl.ds(r0, RPS)])
        pltpu.sync_copy(table_hbm.at[pl.ds(r0, RPS)], acc_sh.at[pl.ds(r0, RPS)])
        pltpu.sync_copy(src_hbm.at[wid], sidx_v)
        pltpu.sync_copy(dst_hbm.at[wid], didx_v)
        plsc.subcore_barrier()

        def step(j, _):
            pltpu.async_copy(tab_sh.at[sidx_v.at[j]], rows_v, sem).wait()
            pltpu.sync_copy(rows_v, acc_sh.at[didx_v.at[j]], add=True)
            return ()

        lax.fori_loop(0, NCHUNK, step, ())
        plsc.subcore_barrier()
        pltpu.sync_copy(acc_sh.at[pl.ds(r0, RPS)], out_hbm.at[c, pl.ds(r0, RPS)])

    return _prop


_prop64 = _make_prop(64)
_prop32 = _make_prop(32)


def _dense1_body(degp_ref, x_ref, w1_ref, dis_ref, h1p_ref):
    deg = degp_ref[0] + degp_ref[1] + 1.0
    dis = lax.rsqrt(deg)
    dis_ref[...] = dis
    h = jnp.dot(x_ref[...], w1_ref[...], preferred_element_type=jnp.float32)
    hp = h * dis[:N, :]
    h1p_ref[...] = jnp.concatenate(
        [hp, jnp.zeros((N_PAD - N, hp.shape[1]), jnp.float32)], axis=0)


def _dense2_body(parts_ref, h1p_ref, dis_ref, w2_ref, b1_ref, h2p_ref):
    acc = parts_ref[0] + parts_ref[1] - h1p_ref[...]
    dis = dis_ref[...]
    h1 = jnp.maximum(dis * acc + b1_ref[...], 0.0)
    h2p_ref[...] = jnp.dot(h1, w2_ref[...], preferred_element_type=jnp.float32) * dis


def _dense3_body(parts_ref, h2p_ref, dis_ref, w3_ref, b2_ref, b3_ref, out_ref):
    acc = parts_ref[0] + parts_ref[1] - h2p_ref[...]
    dis = dis_ref[...]
    h2 = jnp.maximum(dis * acc + b2_ref[...], 0.0)
    out_ref[...] = jnp.dot(h2, w3_ref[...], preferred_element_type=jnp.float32) + b3_ref[...]


def _dense1(deg_parts, x, W1):
    return pl.pallas_call(
        _dense1_body,
        out_shape=(jax.ShapeDtypeStruct((N_PAD, 1), jnp.float32),
                   jax.ShapeDtypeStruct((N_PAD, 64), jnp.float32)),
    )(deg_parts, x, W1)


def _dense2(parts, h1p, dis, W2, b1):
    return pl.pallas_call(
        _dense2_body,
        out_shape=jax.ShapeDtypeStruct((N_PAD, 32), jnp.float32),
    )(parts, h1p, dis, W2, b1)


def _dense3(parts, h2p, dis, W3, b2, b3):
    return pl.pallas_call(
        _dense3_body,
        out_shape=jax.ShapeDtypeStruct((N_PAD, 1), jnp.float32),
    )(parts, h2p, dis, W3, b2, b3)


def kernel(x, edge_index, W1, b1, W2, b2, W3, b3):
    ei = edge_index.astype(jnp.int32)
    src = ei[0].reshape(NW, NCHUNK, CHUNK)
    dst = ei[1].reshape(NW, NCHUNK, CHUNK)

    deg_parts = _deg(dst).reshape(NC, N_PAD, 1)
    dis, h1p = _dense1(deg_parts, x, W1)
    parts1 = _prop64(src, dst, h1p)
    h2p = _dense2(parts1, h1p, dis, W2, b1.reshape(1, -1))
    parts2 = _prop32(src, dst, h2p)
    out = _dense3(parts2, h2p, dis, W3, b2.reshape(1, -1), b3.reshape(1, 1))
    return out[:N]


# speedup vs baseline: 32.5904x; 32.5904x over previous
import functools

import jax
import jax.numpy as jnp
from jax import lax
from jax.experimental import pallas as pl
from jax.experimental.pallas import tpu as pltpu
from jax.experimental.pallas import tpu_sc as plsc

N = 10000
E = 320000
NC = 2
NS = 16
NW = NC * NS
N_PAD = 10240
RPS = N_PAD // NS
EPW = E // NW
CHUNK = 125
NCHUNK = EPW // CHUNK

_mesh = lambda: plsc.VectorSubcoreMesh(
    core_axis_name="c", subcore_axis_name="s", num_cores=NC, num_subcores=NS)


@functools.cache
def _make_deg():
    return functools.partial(
        pl.kernel,
        out_type=jax.ShapeDtypeStruct((NW, N_PAD), jnp.float32),
        mesh=_mesh(),
        compiler_params=pltpu.CompilerParams(needs_layout_passes=False, use_tc_tiling_on_sc=False),
        scratch_types=[
            pltpu.VMEM((N_PAD,), jnp.float32),
            pltpu.VMEM((EPW,), jnp.int32),
        ],
    )(_deg_body)


def _deg_body(dst_hbm, out_hbm, hist_v, didx_v):
    c = lax.axis_index("c")
    s = lax.axis_index("s")
    wid = c * NS + s

    def zstep(k, _):
        hist_v[pl.ds(k * 16, 16)] = jnp.zeros((16,), jnp.float32)
        return ()

    lax.fori_loop(0, N_PAD // 16, zstep, ())
    pltpu.sync_copy(dst_hbm.at[wid], didx_v)
    ones = jnp.ones((16,), jnp.float32)

    def step(k, _):
        idx = didx_v[pl.ds(k * 16, 16)]
        plsc.addupdate_scatter(hist_v, [idx], ones)
        return ()

    lax.fori_loop(0, EPW // 16, step, ())
    pltpu.sync_copy(hist_v, out_hbm.at[wid])


@functools.cache
def _make_prop(D):
    @functools.partial(
        pl.kernel,
        out_type=jax.ShapeDtypeStruct((NC, N_PAD, D), jnp.float32),
        mesh=_mesh(),
        compiler_params=pltpu.CompilerParams(needs_layout_passes=False, use_tc_tiling_on_sc=False),
        scratch_types=[
            pltpu.VMEM_SHARED((N_PAD, D), jnp.float32),
            pltpu.VMEM((NCHUNK, CHUNK), jnp.int32),
            pltpu.VMEM((NCHUNK, CHUNK), jnp.int32),
            pltpu.VMEM((CHUNK, D), jnp.float32),
            pltpu.SemaphoreType.DMA,
        ],
    )
    def _prop(src_hbm, dst_hbm, table_hbm, out_hbm,
              acc_sh, sidx_v, didx_v, rows_v, sem):
        c = lax.axis_index("c")
        s = lax.axis_index("s")
        wid = c * NS + s
        r0 = s * RPS
        pltpu.sync_copy(table_hbm.at[pl.ds(r0, RPS)], acc_sh.at[pl.ds(r0, RPS)])
        pltpu.sync_copy(src_hbm.at[wid], sidx_v)
        pltpu.sync_copy(dst_hbm.at[wid], didx_v)
        plsc.subcore_barrier()

        def step(j, _):
            pltpu.async_copy(table_hbm.at[sidx_v.at[j]], rows_v, sem).wait()
            pltpu.sync_copy(rows_v, acc_sh.at[didx_v.at[j]], add=True)
            return ()

        lax.fori_loop(0, NCHUNK, step, ())
        plsc.subcore_barrier()
        pltpu.sync_copy(acc_sh.at[pl.ds(r0, RPS)], out_hbm.at[c, pl.ds(r0, RPS)])

    return _prop


def _dense1_body(degt_ref, x_ref, w1_ref, dis_ref, h1p_ref):
    deg = jnp.sum(degt_ref[...], axis=1, keepdims=True) + 1.0
    dis = lax.rsqrt(deg)
    dis_ref[...] = dis
    h = jnp.dot(x_ref[...], w1_ref[...], preferred_element_type=jnp.float32)
    hp = h * dis[:N, :]
    h1p_ref[...] = jnp.concatenate(
        [hp, jnp.zeros((N_PAD - N, hp.shape[1]), jnp.float32)], axis=0)


def _dense2_body(parts_ref, h1p_ref, dis_ref, w2_ref, b1_ref, h2p_ref):
    acc = parts_ref[0] + parts_ref[1] - h1p_ref[...]
    dis = dis_ref[...]
    h1 = jnp.maximum(dis * acc + b1_ref[...], 0.0)
    h2p_ref[...] = jnp.dot(h1, w2_ref[...], preferred_element_type=jnp.float32) * dis


def _dense3_body(parts_ref, h2p_ref, dis_ref, w3_ref, b2_ref, b3_ref, out_ref):
    acc = parts_ref[0] + parts_ref[1] - h2p_ref[...]
    dis = dis_ref[...]
    h2 = jnp.maximum(dis * acc + b2_ref[...], 0.0)
    out_ref[...] = jnp.dot(h2, w3_ref[...], preferred_element_type=jnp.float32) + b3_ref[...]


def _dense1(degt, x, W1):
    return pl.pallas_call(
        _dense1_body,
        out_shape=(jax.ShapeDtypeStruct((N_PAD, 1), jnp.float32),
                   jax.ShapeDtypeStruct((N_PAD, 64), jnp.float32)),
    )(degt, x, W1)


def _dense2(parts, h1p, dis, W2, b1):
    return pl.pallas_call(
        _dense2_body,
        out_shape=jax.ShapeDtypeStruct((N_PAD, 32), jnp.float32),
    )(parts, h1p, dis, W2, b1)


def _dense3(parts, h2p, dis, W3, b2, b3):
    return pl.pallas_call(
        _dense3_body,
        out_shape=jax.ShapeDtypeStruct((N_PAD, 1), jnp.float32),
    )(parts, h2p, dis, W3, b2, b3)


def kernel(x, edge_index, W1, b1, W2, b2, W3, b3):
    ei = edge_index.astype(jnp.int32)
    src = ei[0].reshape(NW, NCHUNK, CHUNK)
    dst = ei[1].reshape(NW, NCHUNK, CHUNK)
    dst_flat = ei[1].reshape(NW, EPW)

    deg_parts = _make_deg()(dst_flat)
    dis, h1p = _dense1(deg_parts.T, x, W1)
    parts1 = _make_prop(64)(src, dst, h1p)
    h2p = _dense2(parts1, h1p, dis, W2, b1.reshape(1, -1))
    parts2 = _make_prop(32)(src, dst, h2p)
    out = _dense3(parts2, h2p, dis, W3, b2.reshape(1, -1), b3.reshape(1, 1))
    return out[:N]


# speedup vs baseline: 44.9684x; 1.3798x over previous
import functools

import jax
import jax.numpy as jnp
from jax import lax
from jax.experimental import pallas as pl
from jax.experimental.pallas import tpu as pltpu
from jax.experimental.pallas import tpu_sc as plsc

N = 10000
E = 320000
NC = 2
NS = 16
NW = NC * NS
N_PAD = 10240
RPS = N_PAD // NS
EPW = E // NW
CHUNK = 125
NCHUNK = EPW // CHUNK

_mesh = lambda: plsc.VectorSubcoreMesh(
    core_axis_name="c", subcore_axis_name="s", num_cores=NC, num_subcores=NS)


@functools.cache
def _make_deg():
    return functools.partial(
        pl.kernel,
        out_type=jax.ShapeDtypeStruct((NW, N_PAD), jnp.float32),
        mesh=_mesh(),
        compiler_params=pltpu.CompilerParams(needs_layout_passes=False, use_tc_tiling_on_sc=False),
        scratch_types=[
            pltpu.VMEM((N_PAD,), jnp.float32),
            pltpu.VMEM((EPW,), jnp.int32),
        ],
    )(_deg_body)


def _deg_body(dst_hbm, out_hbm, hist_v, didx_v):
    c = lax.axis_index("c")
    s = lax.axis_index("s")
    wid = c * NS + s

    def zstep(k, _):
        hist_v[pl.ds(k * 16, 16)] = jnp.zeros((16,), jnp.float32)
        return ()

    lax.fori_loop(0, N_PAD // 16, zstep, ())
    pltpu.sync_copy(dst_hbm.at[wid], didx_v)
    ones = jnp.ones((16,), jnp.float32)

    def step(k, _):
        idx = didx_v[pl.ds(k * 16, 16)]
        plsc.addupdate_scatter(hist_v, [idx], ones)
        return ()

    lax.fori_loop(0, EPW // 16, step, ())
    pltpu.sync_copy(hist_v, out_hbm.at[wid])


@functools.cache
def _make_prop(D):
    @functools.partial(
        pl.kernel,
        out_type=jax.ShapeDtypeStruct((NC, N_PAD, D), jnp.float32),
        mesh=_mesh(),
        compiler_params=pltpu.CompilerParams(needs_layout_passes=False, use_tc_tiling_on_sc=False),
        scratch_types=[
            pltpu.VMEM_SHARED((N_PAD, D), jnp.float32),
            pltpu.VMEM((NCHUNK, CHUNK), jnp.int32),
            pltpu.VMEM((NCHUNK, CHUNK), jnp.int32),
            pltpu.VMEM((CHUNK, D), jnp.float32),
            pltpu.VMEM((CHUNK, D), jnp.float32),
            pltpu.SemaphoreType.DMA,
            pltpu.SemaphoreType.DMA,
        ],
    )
    def _prop(src_hbm, dst_hbm, table_hbm, out_hbm,
              acc_sh, sidx_v, didx_v, rows_a, rows_b, sem_a, sem_b):
        c = lax.axis_index("c")
        s = lax.axis_index("s")
        wid = c * NS + s
        r0 = s * RPS
        pltpu.sync_copy(table_hbm.at[pl.ds(r0, RPS)], acc_sh.at[pl.ds(r0, RPS)])
        pltpu.sync_copy(src_hbm.at[wid], sidx_v)
        pltpu.sync_copy(dst_hbm.at[wid], didx_v)
        plsc.subcore_barrier()

        pltpu.async_copy(table_hbm.at[sidx_v.at[0]], rows_a, sem_a)
        pltpu.async_copy(table_hbm.at[sidx_v.at[1]], rows_b, sem_b)

        def step(j2, _):
            j = j2 * 2
            pltpu.make_async_copy(table_hbm.at[sidx_v.at[j]], rows_a, sem_a).wait()
            pltpu.sync_copy(rows_a, acc_sh.at[didx_v.at[j]], add=True)

            @pl.when(j + 2 < NCHUNK)
            def _():
                pltpu.async_copy(table_hbm.at[sidx_v.at[j + 2]], rows_a, sem_a)

            pltpu.make_async_copy(table_hbm.at[sidx_v.at[j + 1]], rows_b, sem_b).wait()
            pltpu.sync_copy(rows_b, acc_sh.at[didx_v.at[j + 1]], add=True)

            @pl.when(j + 3 < NCHUNK)
            def _():
                pltpu.async_copy(table_hbm.at[sidx_v.at[j + 3]], rows_b, sem_b)

            return ()

        lax.fori_loop(0, NCHUNK // 2, step, ())
        plsc.subcore_barrier()
        pltpu.sync_copy(acc_sh.at[pl.ds(r0, RPS)], out_hbm.at[c, pl.ds(r0, RPS)])

    return _prop


def _dense1_body(degt_ref, x_ref, w1_ref, dis_ref, h1p_ref):
    deg = jnp.sum(degt_ref[...], axis=1, keepdims=True) + 1.0
    dis = lax.rsqrt(deg)
    dis_ref[...] = dis
    h = jnp.dot(x_ref[...], w1_ref[...], preferred_element_type=jnp.float32)
    hp = h * dis[:N, :]
    h1p_ref[...] = jnp.concatenate(
        [hp, jnp.zeros((N_PAD - N, hp.shape[1]), jnp.float32)], axis=0)


def _dense2_body(parts_ref, h1p_ref, dis_ref, w2_ref, b1_ref, h2p_ref):
    acc = parts_ref[0] + parts_ref[1] - h1p_ref[...]
    dis = dis_ref[...]
    h1 = jnp.maximum(dis * acc + b1_ref[...], 0.0)
    h2p_ref[...] = jnp.dot(h1, w2_ref[...], preferred_element_type=jnp.float32) * dis


def _dense3_body(parts_ref, h2p_ref, dis_ref, w3_ref, b2_ref, b3_ref, out_ref):
    acc = parts_ref[0] + parts_ref[1] - h2p_ref[...]
    dis = dis_ref[...]
    h2 = jnp.maximum(dis * acc + b2_ref[...], 0.0)
    out_ref[...] = jnp.dot(h2, w3_ref[...], preferred_element_type=jnp.float32) + b3_ref[...]


def _dense1(degt, x, W1):
    return pl.pallas_call(
        _dense1_body,
        out_shape=(jax.ShapeDtypeStruct((N_PAD, 1), jnp.float32),
                   jax.ShapeDtypeStruct((N_PAD, 64), jnp.float32)),
    )(degt, x, W1)


def _dense2(parts, h1p, dis, W2, b1):
    return pl.pallas_call(
        _dense2_body,
        out_shape=jax.ShapeDtypeStruct((N_PAD, 32), jnp.float32),
    )(parts, h1p, dis, W2, b1)


def _dense3(parts, h2p, dis, W3, b2, b3):
    return pl.pallas_call(
        _dense3_body,
        out_shape=jax.ShapeDtypeStruct((N_PAD, 1), jnp.float32),
    )(parts, h2p, dis, W3, b2, b3)


def kernel(x, edge_index, W1, b1, W2, b2, W3, b3):
    ei = edge_index.astype(jnp.int32)
    src = ei[0].reshape(NW, NCHUNK, CHUNK)
    dst = ei[1].reshape(NW, NCHUNK, CHUNK)
    dst_flat = ei[1].reshape(NW, EPW)

    deg_parts = _make_deg()(dst_flat)
    dis, h1p = _dense1(deg_parts.T, x, W1)
    parts1 = _make_prop(64)(src, dst, h1p)
    h2p = _dense2(parts1, h1p, dis, W2, b1.reshape(1, -1))
    parts2 = _make_prop(32)(src, dst, h2p)
    out = _dense3(parts2, h2p, dis, W3, b2.reshape(1, -1), b3.reshape(1, 1))
    return out[:N]


# speedup vs baseline: 52.4658x; 1.1667x over previous
import functools

import jax
import jax.numpy as jnp
from jax import lax
from jax.experimental import pallas as pl
from jax.experimental.pallas import tpu as pltpu
from jax.experimental.pallas import tpu_sc as plsc

N = 10000
E = 320000
NC = 2
NS = 16
NW = NC * NS
N_PAD = 10240
RPS = N_PAD // NS
EPW = E // NW
CHUNK = 125
NCHUNK = EPW // CHUNK

_mesh = lambda: plsc.VectorSubcoreMesh(
    core_axis_name="c", subcore_axis_name="s", num_cores=NC, num_subcores=NS)


@functools.cache
def _make_deg():
    return functools.partial(
        pl.kernel,
        out_type=jax.ShapeDtypeStruct((NW, N_PAD), jnp.float32),
        mesh=_mesh(),
        compiler_params=pltpu.CompilerParams(needs_layout_passes=False, use_tc_tiling_on_sc=False),
        scratch_types=[
            pltpu.VMEM((N_PAD,), jnp.float32),
            pltpu.VMEM((EPW,), jnp.int32),
        ],
    )(_deg_body)


def _deg_body(dst_hbm, out_hbm, hist_v, didx_v):
    c = lax.axis_index("c")
    s = lax.axis_index("s")
    wid = c * NS + s

    def zstep(k, _):
        hist_v[pl.ds(k * 16, 16)] = jnp.zeros((16,), jnp.float32)
        return ()

    lax.fori_loop(0, N_PAD // 16, zstep, ())
    pltpu.sync_copy(dst_hbm.at[wid], didx_v)
    ones = jnp.ones((16,), jnp.float32)

    def step(k, _):
        idx = didx_v[pl.ds(k * 16, 16)]
        plsc.addupdate_scatter(hist_v, [idx], ones)
        return ()

    lax.fori_loop(0, EPW // 16, step, ())
    pltpu.sync_copy(hist_v, out_hbm.at[wid])


NBUF = 4


@functools.cache
def _make_prop(D):
    @functools.partial(
        pl.kernel,
        out_type=jax.ShapeDtypeStruct((NC, N_PAD, D), jnp.float32),
        mesh=_mesh(),
        compiler_params=pltpu.CompilerParams(needs_layout_passes=False, use_tc_tiling_on_sc=False),
        scratch_types=[
            pltpu.VMEM_SHARED((N_PAD, D), jnp.float32),
            pltpu.VMEM((NCHUNK, CHUNK), jnp.int32),
            pltpu.VMEM((NCHUNK, CHUNK), jnp.int32),
        ]
        + [pltpu.VMEM((CHUNK, D), jnp.float32)] * NBUF
        + [pltpu.SemaphoreType.DMA] * (2 * NBUF),
    )
    def _prop(src_hbm, dst_hbm, table_hbm, out_hbm,
              acc_sh, sidx_v, didx_v, *bufs):
        rows = bufs[:NBUF]
        sem_g = bufs[NBUF:2 * NBUF]
        sem_s = bufs[2 * NBUF:]
        c = lax.axis_index("c")
        s = lax.axis_index("s")
        wid = c * NS + s
        r0 = s * RPS
        pltpu.sync_copy(table_hbm.at[pl.ds(r0, RPS)], acc_sh.at[pl.ds(r0, RPS)])
        pltpu.sync_copy(src_hbm.at[wid], sidx_v)
        pltpu.sync_copy(dst_hbm.at[wid], didx_v)
        plsc.subcore_barrier()

        for b in range(NBUF):
            pltpu.async_copy(table_hbm.at[sidx_v.at[b]], rows[b], sem_g[b])

        def step(g, _):
            base = g * NBUF
            for b in range(NBUF):
                j = base + b
                pltpu.make_async_copy(table_hbm.at[sidx_v.at[j]], rows[b], sem_g[b]).wait()
                pltpu.async_copy(rows[b], acc_sh.at[didx_v.at[j]], sem_s[b], add=True)

                @pl.when(j + NBUF < NCHUNK)
                def _():
                    pltpu.make_async_copy(rows[b], acc_sh.at[didx_v.at[j]], sem_s[b]).wait()
                    pltpu.async_copy(table_hbm.at[sidx_v.at[j + NBUF]], rows[b], sem_g[b])

            return ()

        lax.fori_loop(0, NCHUNK // NBUF, step, ())
        for b in range(NBUF):
            j = NCHUNK - NBUF + b
            pltpu.make_async_copy(rows[b], acc_sh.at[didx_v.at[j]], sem_s[b]).wait()
        plsc.subcore_barrier()
        pltpu.sync_copy(acc_sh.at[pl.ds(r0, RPS)], out_hbm.at[c, pl.ds(r0, RPS)])

    return _prop


def _dense1_body(degt_ref, x_ref, w1_ref, dis_ref, h1p_ref):
    deg = jnp.sum(degt_ref[...], axis=1, keepdims=True) + 1.0
    dis = lax.rsqrt(deg)
    dis_ref[...] = dis
    h = jnp.dot(x_ref[...], w1_ref[...], preferred_element_type=jnp.float32)
    hp = h * dis[:N, :]
    h1p_ref[...] = jnp.concatenate(
        [hp, jnp.zeros((N_PAD - N, hp.shape[1]), jnp.float32)], axis=0)


def _dense2_body(parts_ref, h1p_ref, dis_ref, w2_ref, b1_ref, h2p_ref):
    acc = parts_ref[0] + parts_ref[1] - h1p_ref[...]
    dis = dis_ref[...]
    h1 = jnp.maximum(dis * acc + b1_ref[...], 0.0)
    h2p_ref[...] = jnp.dot(h1, w2_ref[...], preferred_element_type=jnp.float32) * dis


def _dense3_body(parts_ref, h2p_ref, dis_ref, w3_ref, b2_ref, b3_ref, out_ref):
    acc = parts_ref[0] + parts_ref[1] - h2p_ref[...]
    dis = dis_ref[...]
    h2 = jnp.maximum(dis * acc + b2_ref[...], 0.0)
    out_ref[...] = jnp.dot(h2, w3_ref[...], preferred_element_type=jnp.float32) + b3_ref[...]


def _dense1(degt, x, W1):
    return pl.pallas_call(
        _dense1_body,
        out_shape=(jax.ShapeDtypeStruct((N_PAD, 1), jnp.float32),
                   jax.ShapeDtypeStruct((N_PAD, 64), jnp.float32)),
    )(degt, x, W1)


def _dense2(parts, h1p, dis, W2, b1):
    return pl.pallas_call(
        _dense2_body,
        out_shape=jax.ShapeDtypeStruct((N_PAD, 32), jnp.float32),
    )(parts, h1p, dis, W2, b1)


def _dense3(parts, h2p, dis, W3, b2, b3):
    return pl.pallas_call(
        _dense3_body,
        out_shape=jax.ShapeDtypeStruct((N_PAD, 1), jnp.float32),
    )(parts, h2p, dis, W3, b2, b3)


def kernel(x, edge_index, W1, b1, W2, b2, W3, b3):
    ei = edge_index.astype(jnp.int32)
    src = ei[0].reshape(NW, NCHUNK, CHUNK)
    dst = ei[1].reshape(NW, NCHUNK, CHUNK)
    dst_flat = ei[1].reshape(NW, EPW)

    deg_parts = _make_deg()(dst_flat)
    dis, h1p = _dense1(deg_parts.T, x, W1)
    parts1 = _make_prop(64)(src, dst, h1p)
    h2p = _dense2(parts1, h1p, dis, W2, b1.reshape(1, -1))
    parts2 = _make_prop(32)(src, dst, h2p)
    out = _dense3(parts2, h2p, dis, W3, b2.reshape(1, -1), b3.reshape(1, 1))
    return out[:N]
